# Initial kernel scaffold; baseline (speedup 1.0000x reference)
import functools

import jax
import jax.numpy as jnp
from jax import lax
from jax.experimental import pallas as pl
from jax.experimental.pallas import tpu as pltpu
from jax.experimental.pallas import tpu_sc as plsc

N = 10000
E = 320000
D = 128
H = 128
NPAT = 16
B = 256
T = 86

NC = 2
NS = 16
NW = NC * NS
PER_TILE = E // NW
CH = 80
NCHUNK = PER_TILE // CH
ROWS_PER_TILE = N // NS

_mesh = plsc.VectorSubcoreMesh(core_axis_name="c", subcore_axis_name="s")


@functools.partial(
    pl.kernel,
    out_type=(
        jax.ShapeDtypeStruct((NC, N, 8), jnp.float32),
        jax.ShapeDtypeStruct((NC, N, 8), jnp.float32),
    ),
    mesh=_mesh,
    scratch_types=[
        pltpu.VMEM_SHARED((N, 8), jnp.float32),
        pltpu.VMEM((NCHUNK, CH), jnp.int32),
        pltpu.VMEM((CH, 8), jnp.float32),
        pltpu.SemaphoreType.DMA,
    ],
)
def _deg_kernel(dst1_hbm, dst2_hbm, ones_hbm, zeros_hbm, out1, out2,
                acc, idx_all, ones_v, ssem):
    c = lax.axis_index("c")
    s = lax.axis_index("s")
    wid = c * NS + s
    pltpu.sync_copy(ones_hbm, ones_v)

    def run_graph(dst_hbm, out_hbm):
        pltpu.sync_copy(zeros_hbm.at[pl.ds(s * ROWS_PER_TILE, ROWS_PER_TILE)],
                        acc.at[pl.ds(s * ROWS_PER_TILE, ROWS_PER_TILE)])
        pltpu.sync_copy(dst_hbm.at[wid], idx_all)
        plsc.subcore_barrier()

        @pl.loop(0, NCHUNK, step=5)
        def _chunks(i):
            for u in range(5):
                pltpu.async_copy(ones_v, acc.at[idx_all.at[i + u]], ssem,
                                 add=True)
            for u in range(5):
                pltpu.make_async_copy(ones_v, acc.at[idx_all.at[i + u]],
                                      ssem).wait()

        plsc.subcore_barrier()
        pltpu.sync_copy(acc.at[pl.ds(s * ROWS_PER_TILE, ROWS_PER_TILE)],
                        out_hbm.at[c, pl.ds(s * ROWS_PER_TILE, ROWS_PER_TILE)])

    run_graph(dst1_hbm, out1)
    run_graph(dst2_hbm, out2)


@functools.partial(
    pl.kernel,
    out_type=(
        jax.ShapeDtypeStruct((NC, N, H), jnp.float32),
        jax.ShapeDtypeStruct((NC, N, H), jnp.float32),
    ),
    mesh=_mesh,
    scratch_types=[
        pltpu.VMEM_SHARED((N, H), jnp.float32),
        pltpu.VMEM((NCHUNK, CH), jnp.int32),
        pltpu.VMEM((NCHUNK, CH), jnp.int32),
        pltpu.VMEM((CH, H), jnp.float32),
        pltpu.VMEM((CH, H), jnp.float32),
        pltpu.SemaphoreType.DMA,
        pltpu.SemaphoreType.DMA,
        pltpu.SemaphoreType.DMA,
        pltpu.SemaphoreType.DMA,
    ],
)
def _edge_kernel(ht1_hbm, src1_hbm, dst1_hbm, ht2_hbm, src2_hbm, dst2_hbm,
                 zeros_hbm, out1, out2,
                 acc, src_all, dst_all, rows0, rows1, g0, g1, s0, s1):
    c = lax.axis_index("c")
    s = lax.axis_index("s")
    wid = c * NS + s
    rows = (rows0, rows1)
    gsem = (g0, g1)
    ssem = (s0, s1)

    def run_graph(ht_hbm, src_hbm, dst_hbm, out_hbm):
        pltpu.sync_copy(zeros_hbm.at[pl.ds(s * ROWS_PER_TILE, ROWS_PER_TILE)],
                        acc.at[pl.ds(s * ROWS_PER_TILE, ROWS_PER_TILE)])
        pltpu.sync_copy(src_hbm.at[wid], src_all)
        pltpu.sync_copy(dst_hbm.at[wid], dst_all)
        plsc.subcore_barrier()

        def gather_start(ii, b):
            pltpu.async_copy(ht_hbm.at[src_all.at[ii]], rows[b], gsem[b])

        def gather_wait(ii, b):
            pltpu.make_async_copy(ht_hbm.at[src_all.at[ii]], rows[b],
                                  gsem[b]).wait()

        def scat_start(ii, b):
            pltpu.async_copy(rows[b], acc.at[dst_all.at[ii]], ssem[b],
                             add=True)

        def scat_wait(ii, b):
            pltpu.make_async_copy(rows[b], acc.at[dst_all.at[ii]],
                                  ssem[b]).wait()

        gather_start(0, 0)
        gather_start(1, 1)

        @pl.loop(0, NCHUNK - 1, step=2)
        def _steady(i):
            for b in range(2):
                j = i + b
                gather_wait(j, b)
                scat_start(j, b)

                @pl.when(j + 2 < NCHUNK)
                def _refill():
                    scat_wait(j, b)
                    gather_start(j + 2, b)

        gather_wait(NCHUNK - 1, 0)
        scat_start(NCHUNK - 1, 0)
        scat_wait(NCHUNK - 2, 1)
        scat_wait(NCHUNK - 1, 0)

        plsc.subcore_barrier()
        pltpu.sync_copy(acc.at[pl.ds(s * ROWS_PER_TILE, ROWS_PER_TILE)],
                        out_hbm.at[c, pl.ds(s * ROWS_PER_TILE, ROWS_PER_TILE)])

    run_graph(ht1_hbm, src1_hbm, dst1_hbm, out1)
    run_graph(ht2_hbm, src2_hbm, dst2_hbm, out2)


RB = 1000
NG = N // RB


def _tc_prep_body(x1_ref, x2_ref, dp1_ref, dp2_ref, wfc_ref, bfc_ref, wg1_ref,
                  ht1_ref, ht2_ref, dinv1_ref, dinv2_ref, weff_s, beff_s):
    i = pl.program_id(0)

    @pl.when(i == 0)
    def _():
        weff_s[...] = jnp.dot(wfc_ref[...], wg1_ref[...],
                              preferred_element_type=jnp.float32)
        beff_s[...] = jnp.dot(bfc_ref[...], wg1_ref[...],
                              preferred_element_type=jnp.float32)

    for x_ref, dp_ref, ht_ref, dinv_ref in (
            (x1_ref, dp1_ref, ht1_ref, dinv1_ref),
            (x2_ref, dp2_ref, ht2_ref, dinv2_ref)):
        dp = dp_ref[...]
        deg = jnp.maximum(dp[0] + dp[1] + 1.0, 1.0)
        dinv = lax.rsqrt(deg)
        dinv_ref[...] = dinv
        h = jnp.dot(x_ref[...], weff_s[...],
                    preferred_element_type=jnp.float32) + beff_s[...]
        ht_ref[...] = dinv[:, 0:1] * h


_tc_prep = pl.pallas_call(
    _tc_prep_body,
    grid=(NG,),
    in_specs=[
        pl.BlockSpec((RB, D), lambda i: (i, 0)),
        pl.BlockSpec((RB, D), lambda i: (i, 0)),
        pl.BlockSpec((2, RB, 8), lambda i: (0, i, 0)),
        pl.BlockSpec((2, RB, 8), lambda i: (0, i, 0)),
        pl.BlockSpec((D, H), lambda i: (0, 0)),
        pl.BlockSpec((1, H), lambda i: (0, 0)),
        pl.BlockSpec((H, H), lambda i: (0, 0)),
    ],
    out_specs=[
        pl.BlockSpec((RB, H), lambda i: (i, 0)),
        pl.BlockSpec((RB, H), lambda i: (i, 0)),
        pl.BlockSpec((RB, 8), lambda i: (i, 0)),
        pl.BlockSpec((RB, 8), lambda i: (i, 0)),
    ],
    out_shape=[
        jax.ShapeDtypeStruct((N, H), jnp.float32),
        jax.ShapeDtypeStruct((N, H), jnp.float32),
        jax.ShapeDtypeStruct((N, 8), jnp.float32),
        jax.ShapeDtypeStruct((N, 8), jnp.float32),
    ],
    scratch_shapes=[
        pltpu.VMEM((D, H), jnp.float32),
        pltpu.VMEM((1, H), jnp.float32),
    ],
)


def _tc_mid_body(p1_ref, p2_ref, ht1a_ref, ht1b_ref, dinv1_ref, dinv2_ref,
                 bg1_ref, wg2_ref, ht2a_ref, ht2b_ref):
    for p_ref, ht1_ref, dinv_ref, ht2_ref in (
            (p1_ref, ht1a_ref, dinv1_ref, ht2a_ref),
            (p2_ref, ht1b_ref, dinv2_ref, ht2b_ref)):
        dinv = dinv_ref[...][:, 0:1]
        p = p_ref[...]
        h1 = dinv * (p[0] + p[1] + ht1_ref[...]) + bg1_ref[...]
        ht2_ref[...] = dinv * jnp.dot(h1, wg2_ref[...],
                                      preferred_element_type=jnp.float32)


_tc_mid = pl.pallas_call(
    _tc_mid_body,
    grid=(NG,),
    in_specs=[
        pl.BlockSpec((2, RB, H), lambda i: (0, i, 0)),
        pl.BlockSpec((2, RB, H), lambda i: (0, i, 0)),
        pl.BlockSpec((RB, H), lambda i: (i, 0)),
        pl.BlockSpec((RB, H), lambda i: (i, 0)),
        pl.BlockSpec((RB, 8), lambda i: (i, 0)),
        pl.BlockSpec((RB, 8), lambda i: (i, 0)),
        pl.BlockSpec((1, H), lambda i: (0, 0)),
        pl.BlockSpec((H, H), lambda i: (0, 0)),
    ],
    out_specs=[
        pl.BlockSpec((RB, H), lambda i: (i, 0)),
        pl.BlockSpec((RB, H), lambda i: (i, 0)),
    ],
    out_shape=[
        jax.ShapeDtypeStruct((N, H), jnp.float32),
        jax.ShapeDtypeStruct((N, H), jnp.float32),
    ],
)


def _tc_pool_body(q_ref, ht2_ref, dinv_ref, bat_ref, bg2_ref, pembt_ref,
                  outsum_ref, pool_ref, acc_out, acc_pool):
    i = pl.program_id(1)

    @pl.when(i == 0)
    def _():
        acc_out[...] = jnp.zeros_like(acc_out)
        acc_pool[...] = jnp.zeros_like(acc_pool)

    q = q_ref[...]
    dinv = dinv_ref[...][0][:, 0:1]
    h2 = dinv * (q[0, 0] + q[0, 1] + ht2_ref[...][0]) + bg2_ref[...]
    scores = jnp.dot(h2, pembt_ref[...],
                     preferred_element_type=jnp.float32)
    m = jnp.max(scores, axis=-1, keepdims=True)
    e = jnp.exp(scores - m)
    a = e / jnp.sum(e, axis=-1, keepdims=True)

    bt = bat_ref[...][0]
    rowid = lax.broadcasted_iota(jnp.int32, (B, RB), 0)
    onehot_t = (rowid == bt).astype(jnp.float32)

    acc_out[...] += jnp.dot(onehot_t, h2, preferred_element_type=jnp.float32)
    for p in range(NPAT):
        acc_pool[p] += jnp.dot(onehot_t, a[:, p:p + 1] * h2,
                               preferred_element_type=jnp.float32)

    @pl.when(i == NG - 1)
    def _():
        outsum_ref[...] = acc_out[...][None]
        pool_ref[...] = acc_pool[...][None]


_tc_pool = pl.pallas_call(
    _tc_pool_body,
    grid=(2, NG),
    in_specs=[
        pl.BlockSpec((1, 2, RB, H), lambda g, i: (g, 0, i, 0)),
        pl.BlockSpec((1, RB, H), lambda g, i: (g, i, 0)),
        pl.BlockSpec((1, RB, 8), lambda g, i: (g, i, 0)),
        pl.BlockSpec((1, 1, RB), lambda g, i: (g * NG + i, 0, 0)),
        pl.BlockSpec((1, H), lambda g, i: (0, 0)),
        pl.BlockSpec((H, NPAT), lambda g, i: (0, 0)),
    ],
    out_specs=[
        pl.BlockSpec((1, B, H), lambda g, i: (g, 0, 0)),
        pl.BlockSpec((1, NPAT, B, H), lambda g, i: (g, 0, 0, 0)),
    ],
    out_shape=[
        jax.ShapeDtypeStruct((2, B, H), jnp.float32),
        jax.ShapeDtypeStruct((2, NPAT, B, H), jnp.float32),
    ],
    scratch_shapes=[
        pltpu.VMEM((B, H), jnp.float32),
        pltpu.VMEM((NPAT, B, H), jnp.float32),
    ],
)


def _tc_final_body(outsum_ref, pool_ref, pemb3_ref, ddi_ref,
                   w0a_ref, w0b_ref, w0c_ref, w0d_ref, bm0_ref,
                   wm1_ref, bm1_ref, wm2_ref, bm2_ref, wout_ref, bout_ref,
                   score_ref):
    pemb = pemb3_ref[...]
    pools = []
    for g in range(2):
        po = pool_ref[...][g] + pemb
        nsq = jnp.sum(po * po, axis=-1, keepdims=True)
        den = jnp.maximum(jnp.sqrt(nsq), 1e-12)
        pools.append(po / den)
    p1n, p2n = pools

    pieces = []
    for p in range(NPAT):
        pieces.append(jnp.sum(p2n * p1n[p][None], axis=-1))
    sim_t = jnp.concatenate(pieces, axis=0)

    ddi = ddi_ref[...][0]
    tid = lax.broadcasted_iota(jnp.int32, (B, T), 1)
    onehot = (tid == ddi[:, None]).astype(jnp.float32)

    outs = outsum_ref[...]
    h = (jnp.dot(outs[0], w0a_ref[...], preferred_element_type=jnp.float32)
         + jnp.dot(outs[1], w0b_ref[...], preferred_element_type=jnp.float32)
         + lax.dot_general(sim_t, w0c_ref[...],
                           (((0,), (0,)), ((), ())),
                           preferred_element_type=jnp.float32)
         + jnp.dot(onehot, w0d_ref[...], preferred_element_type=jnp.float32)
         + bm0_ref[...])
    h = jnp.maximum(jnp.dot(h, wm1_ref[...],
                            preferred_element_type=jnp.float32)
                    + bm1_ref[...], 0.0)
    h = jnp.maximum(jnp.dot(h, wm2_ref[...],
                            preferred_element_type=jnp.float32)
                    + bm2_ref[...], 0.0)
    score_ref[...] = jnp.dot(h, wout_ref[...],
                             preferred_element_type=jnp.float32) + bout_ref[...]


_tc_final = pl.pallas_call(
    _tc_final_body,
    out_shape=jax.ShapeDtypeStruct((B, 1), jnp.float32),
)


def kernel(x1, edge_index1, batch1, x2, edge_index2, batch2, ddi_type,
           W_fc, b_fc, W_g1, b_g1, W_g2, b_g2, P_emb,
           W_m0, b_m0, W_m1, b_m1, W_m2, b_m2, W_out, b_out):
    f32 = jnp.float32
    src1 = edge_index1[0].reshape(NW, NCHUNK, CH)
    dst1 = edge_index1[1].reshape(NW, NCHUNK, CH)
    src2 = edge_index2[0].reshape(NW, NCHUNK, CH)
    dst2 = edge_index2[1].reshape(NW, NCHUNK, CH)

    ones8 = jnp.ones((CH, 8), f32)
    zeros8 = jnp.zeros((N, 8), f32)
    zerosNH = jnp.zeros((N, H), f32)

    dp1, dp2 = _deg_kernel(dst1, dst2, ones8, zeros8)

    ht1_1, ht1_2, dinv1, dinv2 = _tc_prep(
        x1, x2, dp1, dp2, W_fc, b_fc.reshape(1, H), W_g1)

    p1, p2 = _edge_kernel(ht1_1, src1, dst1, ht1_2, src2, dst2, zerosNH)

    ht2_1, ht2_2 = _tc_mid(p1, p2, ht1_1, ht1_2, dinv1, dinv2,
                           b_g1.reshape(1, H), W_g2)

    q1, q2 = _edge_kernel(ht2_1, src1, dst1, ht2_2, src2, dst2, zerosNH)

    batr = jnp.stack([batch1, batch2]).reshape(2 * NG, 1, RB)
    outsum, pool = _tc_pool(
        jnp.stack([q1, q2]), jnp.stack([ht2_1, ht2_2]),
        jnp.stack([dinv1, dinv2]), batr,
        b_g2.reshape(1, H), P_emb.T)

    score = _tc_final(
        outsum, pool, P_emb[:, None, :], ddi_type.reshape(1, B),
        W_m0[0:H], W_m0[H:2 * H], W_m0[2 * H:2 * H + NPAT * NPAT],
        W_m0[2 * H + NPAT * NPAT:], b_m0.reshape(1, H),
        W_m1, b_m1.reshape(1, H), W_m2, b_m2.reshape(1, H),
        W_out, b_out.reshape(1, 1))
    return score[:, 0]


# speedup vs baseline: 19.7734x; 19.7734x over previous
import functools

import jax
import jax.numpy as jnp
from jax import lax
from jax.experimental import pallas as pl
from jax.experimental.pallas import tpu as pltpu
from jax.experimental.pallas import tpu_sc as plsc

N = 10000
E = 320000
D = 128
H = 128
NPAT = 16
B = 256
T = 86

NC = 2
NS = 16
NW = NC * NS
PER_TILE = E // NW
CH = 40
NCHUNK = PER_TILE // CH
SLAB = 624
TAIL_START = SLAB * NS
TAIL = N - TAIL_START

NPAD1 = 10240
SLAB1 = NPAD1 // NS


def _tile_rows_copy(src_ref, dst_ref, s):
    start = pl.multiple_of(s * SLAB, 8)
    pltpu.sync_copy(src_ref.at[pl.ds(start, SLAB)],
                    dst_ref.at[pl.ds(start, SLAB)])

    @pl.when(s == NS - 1)
    def _():
        pltpu.sync_copy(src_ref.at[pl.ds(TAIL_START, TAIL)],
                        dst_ref.at[pl.ds(TAIL_START, TAIL)])


def _tile_rows_copy_1d(src_ref, dst_ref, s):
    start = pl.multiple_of(s * SLAB1, 128)
    pltpu.sync_copy(src_ref.at[pl.ds(start, SLAB1)],
                    dst_ref.at[pl.ds(start, SLAB1)])


def _deg_body(dst1_hbm, dst2_hbm, ones_hbm, zeros_hbm, out1, out2,
              acc, idx_all, ones_v, ssem):
    c = lax.axis_index("c")
    s = lax.axis_index("s")
    wid = c * NS + s
    pltpu.sync_copy(ones_hbm, ones_v)

    def run_graph(dst_hbm, out_hbm):
        _tile_rows_copy_1d(zeros_hbm, acc, s)
        pltpu.sync_copy(dst_hbm.at[wid], idx_all)
        plsc.subcore_barrier()

        @pl.loop(0, NCHUNK, step=5)
        def _chunks(i):
            for u in range(5):
                pltpu.async_copy(ones_v, acc.at[idx_all.at[i + u]], ssem,
                                 add=True)
            for u in range(5):
                pltpu.make_async_copy(ones_v, acc.at[idx_all.at[i + u]],
                                      ssem).wait()

        plsc.subcore_barrier()
        _tile_rows_copy_1d(acc, out_hbm.at[c], s)

    run_graph(dst1_hbm, out1)
    run_graph(dst2_hbm, out2)


def _edge_body(ht1_hbm, e1_hbm, ht2_hbm, e2_hbm, zeros_hbm, out1, out2,
               acc, i0, i1, i2, i3, rows0, rows1,
               is0, is1, is2, is3, g0, g1, s0, s1):
    c = lax.axis_index("c")
    s = lax.axis_index("s")
    wid = c * NS + s
    idxb = (i0, i1, i2, i3)
    isem = (is0, is1, is2, is3)
    rows = (rows0, rows1)
    gsem = (g0, g1)
    ssem = (s0, s1)

    def run_graph(ht_hbm, e_hbm, out_hbm):
        _tile_rows_copy(zeros_hbm, acc, s)
        plsc.subcore_barrier()

        def idx_start(j, q):
            pltpu.async_copy(e_hbm.at[wid, j], idxb[q], isem[q])

        def idx_wait(j, q):
            pltpu.make_async_copy(e_hbm.at[wid, j], idxb[q], isem[q]).wait()

        def gather_start(b, q):
            pltpu.async_copy(ht_hbm.at[idxb[q].at[0]], rows[b], gsem[b])

        def gather_wait(b, q):
            pltpu.make_async_copy(ht_hbm.at[idxb[q].at[0]], rows[b],
                                  gsem[b]).wait()

        def scat_start(b, q):
            pltpu.async_copy(rows[b], acc.at[idxb[q].at[1]], ssem[b],
                             add=True)

        def scat_wait(b, q):
            pltpu.make_async_copy(rows[b], acc.at[idxb[q].at[1]],
                                  ssem[b]).wait()

        for q in range(4):
            idx_start(q, q)
        idx_wait(0, 0)
        gather_start(0, 0)
        idx_wait(1, 1)
        gather_start(1, 1)

        @pl.loop(0, NCHUNK - 2, step=4)
        def _steady(i):
            for u in range(4):
                j = i + u
                b = u % 2
                gather_wait(b, u)
                scat_start(b, u)

                @pl.when(j + 2 < NCHUNK)
                def _refill():
                    scat_wait(b, u)
                    idx_wait(j + 2, (u + 2) % 4)
                    gather_start(b, (u + 2) % 4)

                @pl.when(j + 4 < NCHUNK)
                def _reload():
                    idx_start(j + 4, u)

        for (b, q) in ((0, 0), (1, 1)):
            gather_wait(b, q)
            scat_start(b, q)
        scat_wait(0, 0)
        scat_wait(1, 1)

        plsc.subcore_barrier()
        _tile_rows_copy(acc, out_hbm.at[c], s)

    run_graph(ht1_hbm, e1_hbm, out1)
    run_graph(ht2_hbm, e2_hbm, out2)


@functools.lru_cache(maxsize=None)
def _sc_kernels():
    mesh = plsc.VectorSubcoreMesh(core_axis_name="c", subcore_axis_name="s",
                                  num_cores=NC, num_subcores=NS)
    deg = pl.kernel(
        _deg_body,
        out_type=(
            jax.ShapeDtypeStruct((NC, NPAD1), jnp.float32),
            jax.ShapeDtypeStruct((NC, NPAD1), jnp.float32),
        ),
        mesh=mesh,
        scratch_types=[
            pltpu.VMEM_SHARED((NPAD1,), jnp.float32),
            pltpu.VMEM((NCHUNK, CH), jnp.int32),
            pltpu.VMEM((CH,), jnp.float32),
            pltpu.SemaphoreType.DMA,
        ],
    )
    edge = pl.kernel(
        _edge_body,
        out_type=(
            jax.ShapeDtypeStruct((NC, N, H), jnp.float32),
            jax.ShapeDtypeStruct((NC, N, H), jnp.float32),
        ),
        mesh=mesh,
        scratch_types=[
            pltpu.VMEM_SHARED((N, H), jnp.float32),
            pltpu.VMEM((2, CH), jnp.int32),
            pltpu.VMEM((2, CH), jnp.int32),
            pltpu.VMEM((2, CH), jnp.int32),
            pltpu.VMEM((2, CH), jnp.int32),
            pltpu.VMEM((CH, H), jnp.float32),
            pltpu.VMEM((CH, H), jnp.float32),
            pltpu.SemaphoreType.DMA,
            pltpu.SemaphoreType.DMA,
            pltpu.SemaphoreType.DMA,
            pltpu.SemaphoreType.DMA,
            pltpu.SemaphoreType.DMA,
            pltpu.SemaphoreType.DMA,
            pltpu.SemaphoreType.DMA,
            pltpu.SemaphoreType.DMA,
        ],
    )
    return deg, edge


RB = 1000
NG = N // RB


def _tc_prep_body(x1_ref, x2_ref, dp1_ref, dp2_ref, wfc_ref, bfc_ref, wg1_ref,
                  ht1_ref, ht2_ref, dinv1_ref, dinv2_ref, weff_s, beff_s):
    i = pl.program_id(0)

    @pl.when(i == 0)
    def _():
        weff_s[...] = jnp.dot(wfc_ref[...], wg1_ref[...],
                              preferred_element_type=jnp.float32)
        beff_s[...] = jnp.dot(bfc_ref[...], wg1_ref[...],
                              preferred_element_type=jnp.float32)

    for x_ref, dp_ref, ht_ref, dinv_ref in (
            (x1_ref, dp1_ref, ht1_ref, dinv1_ref),
            (x2_ref, dp2_ref, ht2_ref, dinv2_ref)):
        dp = dp_ref[...]
        deg = jnp.maximum(dp[:, 0:1] + dp[:, 1:2] + 1.0, 1.0)
        dinv = lax.rsqrt(deg)
        dinv_ref[...] = jnp.broadcast_to(dinv, (RB, 8))
        h = jnp.dot(x_ref[...], weff_s[...],
                    preferred_element_type=jnp.float32) + beff_s[...]
        ht_ref[...] = dinv * h


_tc_prep = pl.pallas_call(
    _tc_prep_body,
    grid=(NG,),
    in_specs=[
        pl.BlockSpec((RB, D), lambda i: (i, 0)),
        pl.BlockSpec((RB, D), lambda i: (i, 0)),
        pl.BlockSpec((RB, NC), lambda i: (i, 0)),
        pl.BlockSpec((RB, NC), lambda i: (i, 0)),
        pl.BlockSpec((D, H), lambda i: (0, 0)),
        pl.BlockSpec((1, H), lambda i: (0, 0)),
        pl.BlockSpec((H, H), lambda i: (0, 0)),
    ],
    out_specs=[
        pl.BlockSpec((RB, H), lambda i: (i, 0)),
        pl.BlockSpec((RB, H), lambda i: (i, 0)),
        pl.BlockSpec((RB, 8), lambda i: (i, 0)),
        pl.BlockSpec((RB, 8), lambda i: (i, 0)),
    ],
    out_shape=[
        jax.ShapeDtypeStruct((N, H), jnp.float32),
        jax.ShapeDtypeStruct((N, H), jnp.float32),
        jax.ShapeDtypeStruct((N, 8), jnp.float32),
        jax.ShapeDtypeStruct((N, 8), jnp.float32),
    ],
    scratch_shapes=[
        pltpu.VMEM((D, H), jnp.float32),
        pltpu.VMEM((1, H), jnp.float32),
    ],
)


def _tc_mid_body(p1_ref, p2_ref, ht1a_ref, ht1b_ref, dinv1_ref, dinv2_ref,
                 bg1_ref, wg2_ref, ht2a_ref, ht2b_ref):
    for p_ref, ht1_ref, dinv_ref, ht2_ref in (
            (p1_ref, ht1a_ref, dinv1_ref, ht2a_ref),
            (p2_ref, ht1b_ref, dinv2_ref, ht2b_ref)):
        dinv = dinv_ref[...][:, 0:1]
        p = p_ref[...]
        h1 = dinv * (p[0] + p[1] + ht1_ref[...]) + bg1_ref[...]
        ht2_ref[...] = dinv * jnp.dot(h1, wg2_ref[...],
                                      preferred_element_type=jnp.float32)


_tc_mid = pl.pallas_call(
    _tc_mid_body,
    grid=(NG,),
    in_specs=[
        pl.BlockSpec((2, RB, H), lambda i: (0, i, 0)),
        pl.BlockSpec((2, RB, H), lambda i: (0, i, 0)),
        pl.BlockSpec((RB, H), lambda i: (i, 0)),
        pl.BlockSpec((RB, H), lambda i: (i, 0)),
        pl.BlockSpec((RB, 8), lambda i: (i, 0)),
        pl.BlockSpec((RB, 8), lambda i: (i, 0)),
        pl.BlockSpec((1, H), lambda i: (0, 0)),
        pl.BlockSpec((H, H), lambda i: (0, 0)),
    ],
    out_specs=[
        pl.BlockSpec((RB, H), lambda i: (i, 0)),
        pl.BlockSpec((RB, H), lambda i: (i, 0)),
    ],
    out_shape=[
        jax.ShapeDtypeStruct((N, H), jnp.float32),
        jax.ShapeDtypeStruct((N, H), jnp.float32),
    ],
)


def _tc_pool_body(q_ref, ht2_ref, dinv_ref, bat_ref, bg2_ref, pembt_ref,
                  outsum_ref, pool_ref, acc_out, acc_pool):
    i = pl.program_id(1)

    @pl.when(i == 0)
    def _():
        acc_out[...] = jnp.zeros_like(acc_out)
        acc_pool[...] = jnp.zeros_like(acc_pool)

    q = q_ref[...]
    dinv = dinv_ref[...][0][:, 0:1]
    h2 = dinv * (q[0, 0] + q[0, 1] + ht2_ref[...][0]) + bg2_ref[...]
    scores = jnp.dot(h2, pembt_ref[...],
                     preferred_element_type=jnp.float32)
    m = jnp.max(scores, axis=-1, keepdims=True)
    e = jnp.exp(scores - m)
    a = e / jnp.sum(e, axis=-1, keepdims=True)

    bt = bat_ref[...][0]
    rowid = lax.broadcasted_iota(jnp.int32, (B, RB), 0)
    onehot_t = (rowid == bt).astype(jnp.float32)

    acc_out[...] += jnp.dot(onehot_t, h2, preferred_element_type=jnp.float32)
    for p in range(NPAT):
        acc_pool[p] += jnp.dot(onehot_t, a[:, p:p + 1] * h2,
                               preferred_element_type=jnp.float32)

    @pl.when(i == NG - 1)
    def _():
        outsum_ref[...] = acc_out[...][None]
        pool_ref[...] = acc_pool[...][None]


_tc_pool = pl.pallas_call(
    _tc_pool_body,
    grid=(2, NG),
    in_specs=[
        pl.BlockSpec((1, 2, RB, H), lambda g, i: (g, 0, i, 0)),
        pl.BlockSpec((1, RB, H), lambda g, i: (g, i, 0)),
        pl.BlockSpec((1, RB, 8), lambda g, i: (g, i, 0)),
        pl.BlockSpec((1, 1, RB), lambda g, i: (g * NG + i, 0, 0)),
        pl.BlockSpec((1, H), lambda g, i: (0, 0)),
        pl.BlockSpec((H, NPAT), lambda g, i: (0, 0)),
    ],
    out_specs=[
        pl.BlockSpec((1, B, H), lambda g, i: (g, 0, 0)),
        pl.BlockSpec((1, NPAT, B, H), lambda g, i: (g, 0, 0, 0)),
    ],
    out_shape=[
        jax.ShapeDtypeStruct((2, B, H), jnp.float32),
        jax.ShapeDtypeStruct((2, NPAT, B, H), jnp.float32),
    ],
    scratch_shapes=[
        pltpu.VMEM((B, H), jnp.float32),
        pltpu.VMEM((NPAT, B, H), jnp.float32),
    ],
)


def _tc_final_body(outsum_ref, pool_ref, pemb3_ref, ddi_ref,
                   w0a_ref, w0b_ref, w0c_ref, w0d_ref, bm0_ref,
                   wm1_ref, bm1_ref, wm2_ref, bm2_ref, wout_ref, bout_ref,
                   score_ref):
    pemb = pemb3_ref[...]
    pools = []
    for g in range(2):
        po = pool_ref[...][g] + pemb
        nsq = jnp.sum(po * po, axis=-1, keepdims=True)
        den = jnp.maximum(jnp.sqrt(nsq), 1e-12)
        pools.append(po / den)
    p1n, p2n = pools

    pieces = []
    for p in range(NPAT):
        pieces.append(jnp.sum(p2n * p1n[p][None], axis=-1))
    sim_t = jnp.concatenate(pieces, axis=0)

    ddi = ddi_ref[...][0]
    tid = lax.broadcasted_iota(jnp.int32, (B, T), 1)
    onehot = (tid == ddi[:, None]).astype(jnp.float32)

    outs = outsum_ref[...]
    h = (jnp.dot(outs[0], w0a_ref[...], preferred_element_type=jnp.float32)
         + jnp.dot(outs[1], w0b_ref[...], preferred_element_type=jnp.float32)
         + lax.dot_general(sim_t, w0c_ref[...],
                           (((0,), (0,)), ((), ())),
                           preferred_element_type=jnp.float32)
         + jnp.dot(onehot, w0d_ref[...], preferred_element_type=jnp.float32)
         + bm0_ref[...])
    h = jnp.maximum(jnp.dot(h, wm1_ref[...],
                            preferred_element_type=jnp.float32)
                    + bm1_ref[...], 0.0)
    h = jnp.maximum(jnp.dot(h, wm2_ref[...],
                            preferred_element_type=jnp.float32)
                    + bm2_ref[...], 0.0)
    score_ref[...] = jnp.dot(h, wout_ref[...],
                             preferred_element_type=jnp.float32) + bout_ref[...]


_tc_final = pl.pallas_call(
    _tc_final_body,
    out_shape=jax.ShapeDtypeStruct((B, 1), jnp.float32),
)


def kernel(x1, edge_index1, batch1, x2, edge_index2, batch2, ddi_type,
           W_fc, b_fc, W_g1, b_g1, W_g2, b_g2, P_emb,
           W_m0, b_m0, W_m1, b_m1, W_m2, b_m2, W_out, b_out):
    f32 = jnp.float32
    dst1 = edge_index1[1].reshape(NW, NCHUNK, CH)
    dst2 = edge_index2[1].reshape(NW, NCHUNK, CH)
    e1 = edge_index1.reshape(2, NW, NCHUNK, CH).transpose(1, 2, 0, 3)
    e2 = edge_index2.reshape(2, NW, NCHUNK, CH).transpose(1, 2, 0, 3)

    ones1 = jnp.ones((CH,), f32)
    zeros1 = jnp.zeros((NPAD1,), f32)
    zerosNH = jnp.zeros((N, H), f32)

    deg_k, edge_k = _sc_kernels()
    dp1, dp2 = deg_k(dst1, dst2, ones1, zeros1)

    ht1_1, ht1_2, dinv1, dinv2 = _tc_prep(
        x1, x2, dp1.T[:N], dp2.T[:N], W_fc, b_fc.reshape(1, H), W_g1)

    p1, p2 = edge_k(ht1_1, e1, ht1_2, e2, zerosNH)

    ht2_1, ht2_2 = _tc_mid(p1, p2, ht1_1, ht1_2, dinv1, dinv2,
                           b_g1.reshape(1, H), W_g2)

    q1, q2 = edge_k(ht2_1, e1, ht2_2, e2, zerosNH)

    batr = jnp.stack([batch1, batch2]).reshape(2 * NG, 1, RB)
    outsum, pool = _tc_pool(
        jnp.stack([q1, q2]), jnp.stack([ht2_1, ht2_2]),
        jnp.stack([dinv1, dinv2]), batr,
        b_g2.reshape(1, H), P_emb.T)

    score = _tc_final(
        outsum, pool, P_emb[:, None, :], ddi_type.reshape(1, B),
        W_m0[0:H], W_m0[H:2 * H], W_m0[2 * H:2 * H + NPAT * NPAT],
        W_m0[2 * H + NPAT * NPAT:], b_m0.reshape(1, H),
        W_m1, b_m1.reshape(1, H), W_m2, b_m2.reshape(1, H),
        W_out, b_out.reshape(1, 1))
    return score[:, 0]


# speedup vs baseline: 25.6588x; 1.2976x over previous
import functools

import jax
import jax.numpy as jnp
from jax import lax
from jax.experimental import pallas as pl
from jax.experimental.pallas import tpu as pltpu
from jax.experimental.pallas import tpu_sc as plsc

N = 10000
E = 320000
D = 128
H = 128
NPAT = 16
B = 256
T = 86

NC = 2
NS = 16
NW = NC * NS
PER_TILE = E // NW
CH = 80
NCHUNK = PER_TILE // CH
SLAB = 624
TAIL_START = SLAB * NS
TAIL = N - TAIL_START

NPAD1 = 10240
SLAB1 = NPAD1 // NS


def _tile_rows_copy(src_ref, dst_ref, s):
    start = pl.multiple_of(s * SLAB, 8)
    pltpu.sync_copy(src_ref.at[pl.ds(start, SLAB)],
                    dst_ref.at[pl.ds(start, SLAB)])

    @pl.when(s == NS - 1)
    def _():
        pltpu.sync_copy(src_ref.at[pl.ds(TAIL_START, TAIL)],
                        dst_ref.at[pl.ds(TAIL_START, TAIL)])


def _tile_rows_copy_1d(src_ref, dst_ref, s):
    start = pl.multiple_of(s * SLAB1, 128)
    pltpu.sync_copy(src_ref.at[pl.ds(start, SLAB1)],
                    dst_ref.at[pl.ds(start, SLAB1)])


def _deg_body(dst1_hbm, dst2_hbm, ones_hbm, zeros_hbm, out1, out2,
              acc, idx_all, ones_v, ssem):
    c = lax.axis_index("c")
    s = lax.axis_index("s")
    wid = c * NS + s
    pltpu.sync_copy(ones_hbm, ones_v)

    def run_graph(dst_hbm, out_hbm):
        _tile_rows_copy_1d(zeros_hbm, acc, s)
        pltpu.sync_copy(dst_hbm.at[wid], idx_all)
        plsc.subcore_barrier()

        @pl.loop(0, NCHUNK, step=5)
        def _chunks(i):
            for u in range(5):
                pltpu.async_copy(ones_v, acc.at[idx_all.at[i + u]], ssem,
                                 add=True)
            for u in range(5):
                pltpu.make_async_copy(ones_v, acc.at[idx_all.at[i + u]],
                                      ssem).wait()

        plsc.subcore_barrier()
        _tile_rows_copy_1d(acc, out_hbm.at[c], s)

    run_graph(dst1_hbm, out1)
    run_graph(dst2_hbm, out2)


def _edge_body(ht1_hbm, e1_hbm, ht2_hbm, e2_hbm, zeros_hbm, out1, out2,
               acc, i0, i1, i2, i3, d0, d1, r0, r1, r2, r3,
               is0, is1, is2, is3, g0, g1, g2, g3, s0, s1, s2, s3):
    c = lax.axis_index("c")
    s = lax.axis_index("s")
    wid = c * NS + s
    idxb = (i0, i1, i2, i3)
    dsti = (d0, d1)
    rows = (r0, r1, r2, r3)
    isem = (is0, is1, is2, is3)
    gsem = (g0, g1, g2, g3)
    ssem = (s0, s1, s2, s3)

    def run_graph(ht_hbm, e_hbm, out_hbm):
        _tile_rows_copy(zeros_hbm, acc, s)
        plsc.subcore_barrier()

        def idx_start(j, q):
            pltpu.async_copy(e_hbm.at[wid, j], idxb[q], isem[q])

        def idx_wait(j, q):
            pltpu.make_async_copy(e_hbm.at[wid, j], idxb[q], isem[q]).wait()

        def gather_start(u):
            pltpu.async_copy(ht_hbm.at[idxb[u].at[0]], rows[u], gsem[u])

        def gather_wait(u):
            pltpu.make_async_copy(ht_hbm.at[idxb[u].at[0]], rows[u],
                                  gsem[u]).wait()

        def copy_dst(u, w):
            for k in range(CH // 16):
                dsti[w][0, pl.ds(16 * k, 16)] = idxb[u][1, pl.ds(16 * k, 16)]

        def scat_start(u, w):
            pltpu.async_copy(rows[u], acc.at[dsti[w].at[0]], ssem[u],
                             add=True)

        def scat_wait(u, w):
            pltpu.make_async_copy(rows[u], acc.at[dsti[w].at[0]],
                                  ssem[u]).wait()

        for q in range(4):
            idx_start(q, q)
        idx_wait(0, 0)
        gather_start(0)
        idx_wait(1, 1)
        gather_start(1)

        @pl.loop(0, NCHUNK - 1, step=4)
        def _steady(i):
            for u in range(4):
                j = i + u
                w = u % 2
                u2 = (u + 2) % 4
                gather_wait(u)

                @pl.when(j >= 2)
                def _drain_prev():
                    scat_wait(u2, w)

                copy_dst(u, w)
                scat_start(u, w)

                @pl.when(j + 2 < NCHUNK)
                def _refill():
                    idx_wait(j + 2, u2)
                    gather_start(u2)

                @pl.when(j + 4 < NCHUNK)
                def _reload():
                    idx_start(j + 4, u)

        gather_wait(0)
        scat_wait(2, 0)
        copy_dst(0, 0)
        scat_start(0, 0)
        scat_wait(3, 1)
        scat_wait(0, 0)

        plsc.subcore_barrier()
        _tile_rows_copy(acc, out_hbm.at[c], s)

    run_graph(ht1_hbm, e1_hbm, out1)
    run_graph(ht2_hbm, e2_hbm, out2)


@functools.lru_cache(maxsize=None)
def _sc_kernels():
    mesh = plsc.VectorSubcoreMesh(core_axis_name="c", subcore_axis_name="s",
                                  num_cores=NC, num_subcores=NS)
    deg = pl.kernel(
        _deg_body,
        out_type=(
            jax.ShapeDtypeStruct((NC, NPAD1), jnp.float32),
            jax.ShapeDtypeStruct((NC, NPAD1), jnp.float32),
        ),
        mesh=mesh,
        scratch_types=[
            pltpu.VMEM_SHARED((NPAD1,), jnp.float32),
            pltpu.VMEM((NCHUNK, CH), jnp.int32),
            pltpu.VMEM((CH,), jnp.float32),
            pltpu.SemaphoreType.DMA,
        ],
    )
    edge = pl.kernel(
        _edge_body,
        out_type=(
            jax.ShapeDtypeStruct((NC, N, H), jnp.float32),
            jax.ShapeDtypeStruct((NC, N, H), jnp.float32),
        ),
        mesh=mesh,
        scratch_types=[
            pltpu.VMEM_SHARED((N, H), jnp.float32),
            pltpu.VMEM((2, CH), jnp.int32),
            pltpu.VMEM((2, CH), jnp.int32),
            pltpu.VMEM((2, CH), jnp.int32),
            pltpu.VMEM((2, CH), jnp.int32),
            pltpu.VMEM((1, CH), jnp.int32),
            pltpu.VMEM((1, CH), jnp.int32),
            pltpu.VMEM((CH, H), jnp.float32),
            pltpu.VMEM((CH, H), jnp.float32),
            pltpu.VMEM((CH, H), jnp.float32),
            pltpu.VMEM((CH, H), jnp.float32),
        ] + [pltpu.SemaphoreType.DMA] * 12,
    )
    return deg, edge


RB = 1000
NG = N // RB


def _tc_prep_body(x1_ref, x2_ref, dp1_ref, dp2_ref, wfc_ref, bfc_ref, wg1_ref,
                  ht1_ref, ht2_ref, dinv1_ref, dinv2_ref, weff_s, beff_s):
    i = pl.program_id(0)

    @pl.when(i == 0)
    def _():
        weff_s[...] = jnp.dot(wfc_ref[...], wg1_ref[...],
                              preferred_element_type=jnp.float32)
        beff_s[...] = jnp.dot(bfc_ref[...], wg1_ref[...],
                              preferred_element_type=jnp.float32)

    for x_ref, dp_ref, ht_ref, dinv_ref in (
            (x1_ref, dp1_ref, ht1_ref, dinv1_ref),
            (x2_ref, dp2_ref, ht2_ref, dinv2_ref)):
        dp = dp_ref[...]
        deg = jnp.maximum(dp[:, 0:1] + dp[:, 1:2] + 1.0, 1.0)
        dinv = lax.rsqrt(deg)
        dinv_ref[...] = jnp.broadcast_to(dinv, (RB, 8))
        h = jnp.dot(x_ref[...], weff_s[...],
                    preferred_element_type=jnp.float32) + beff_s[...]
        ht_ref[...] = dinv * h


_tc_prep = pl.pallas_call(
    _tc_prep_body,
    grid=(NG,),
    in_specs=[
        pl.BlockSpec((RB, D), lambda i: (i, 0)),
        pl.BlockSpec((RB, D), lambda i: (i, 0)),
        pl.BlockSpec((RB, NC), lambda i: (i, 0)),
        pl.BlockSpec((RB, NC), lambda i: (i, 0)),
        pl.BlockSpec((D, H), lambda i: (0, 0)),
        pl.BlockSpec((1, H), lambda i: (0, 0)),
        pl.BlockSpec((H, H), lambda i: (0, 0)),
    ],
    out_specs=[
        pl.BlockSpec((RB, H), lambda i: (i, 0)),
        pl.BlockSpec((RB, H), lambda i: (i, 0)),
        pl.BlockSpec((RB, 8), lambda i: (i, 0)),
        pl.BlockSpec((RB, 8), lambda i: (i, 0)),
    ],
    out_shape=[
        jax.ShapeDtypeStruct((N, H), jnp.float32),
        jax.ShapeDtypeStruct((N, H), jnp.float32),
        jax.ShapeDtypeStruct((N, 8), jnp.float32),
        jax.ShapeDtypeStruct((N, 8), jnp.float32),
    ],
    scratch_shapes=[
        pltpu.VMEM((D, H), jnp.float32),
        pltpu.VMEM((1, H), jnp.float32),
    ],
)


def _tc_mid_body(p1_ref, p2_ref, ht1a_ref, ht1b_ref, dinv1_ref, dinv2_ref,
                 bg1_ref, wg2_ref, ht2a_ref, ht2b_ref):
    for p_ref, ht1_ref, dinv_ref, ht2_ref in (
            (p1_ref, ht1a_ref, dinv1_ref, ht2a_ref),
            (p2_ref, ht1b_ref, dinv2_ref, ht2b_ref)):
        dinv = dinv_ref[...][:, 0:1]
        p = p_ref[...]
        h1 = dinv * (p[0] + p[1] + ht1_ref[...]) + bg1_ref[...]
        ht2_ref[...] = dinv * jnp.dot(h1, wg2_ref[...],
                                      preferred_element_type=jnp.float32)


_tc_mid = pl.pallas_call(
    _tc_mid_body,
    grid=(NG,),
    in_specs=[
        pl.BlockSpec((2, RB, H), lambda i: (0, i, 0)),
        pl.BlockSpec((2, RB, H), lambda i: (0, i, 0)),
        pl.BlockSpec((RB, H), lambda i: (i, 0)),
        pl.BlockSpec((RB, H), lambda i: (i, 0)),
        pl.BlockSpec((RB, 8), lambda i: (i, 0)),
        pl.BlockSpec((RB, 8), lambda i: (i, 0)),
        pl.BlockSpec((1, H), lambda i: (0, 0)),
        pl.BlockSpec((H, H), lambda i: (0, 0)),
    ],
    out_specs=[
        pl.BlockSpec((RB, H), lambda i: (i, 0)),
        pl.BlockSpec((RB, H), lambda i: (i, 0)),
    ],
    out_shape=[
        jax.ShapeDtypeStruct((N, H), jnp.float32),
        jax.ShapeDtypeStruct((N, H), jnp.float32),
    ],
)


def _tc_pool_body(q_ref, ht2_ref, dinv_ref, bat_ref, bg2_ref, pembt_ref,
                  outsum_ref, pool_ref, acc_out, acc_pool):
    i = pl.program_id(1)

    @pl.when(i == 0)
    def _():
        acc_out[...] = jnp.zeros_like(acc_out)
        acc_pool[...] = jnp.zeros_like(acc_pool)

    q = q_ref[...]
    dinv = dinv_ref[...][0][:, 0:1]
    h2 = dinv * (q[0, 0] + q[0, 1] + ht2_ref[...][0]) + bg2_ref[...]
    scores = jnp.dot(h2, pembt_ref[...],
                     preferred_element_type=jnp.float32)
    m = jnp.max(scores, axis=-1, keepdims=True)
    e = jnp.exp(scores - m)
    a = e / jnp.sum(e, axis=-1, keepdims=True)

    bt = bat_ref[...][0]
    rowid = lax.broadcasted_iota(jnp.int32, (B, RB), 0)
    onehot_t = (rowid == bt).astype(jnp.float32)

    acc_out[...] += jnp.dot(onehot_t, h2, preferred_element_type=jnp.float32)
    for p in range(NPAT):
        acc_pool[p] += jnp.dot(onehot_t, a[:, p:p + 1] * h2,
                               preferred_element_type=jnp.float32)

    @pl.when(i == NG - 1)
    def _():
        outsum_ref[...] = acc_out[...][None]
        pool_ref[...] = acc_pool[...][None]


_tc_pool = pl.pallas_call(
    _tc_pool_body,
    grid=(2, NG),
    in_specs=[
        pl.BlockSpec((1, 2, RB, H), lambda g, i: (g, 0, i, 0)),
        pl.BlockSpec((1, RB, H), lambda g, i: (g, i, 0)),
        pl.BlockSpec((1, RB, 8), lambda g, i: (g, i, 0)),
        pl.BlockSpec((1, 1, RB), lambda g, i: (g * NG + i, 0, 0)),
        pl.BlockSpec((1, H), lambda g, i: (0, 0)),
        pl.BlockSpec((H, NPAT), lambda g, i: (0, 0)),
    ],
    out_specs=[
        pl.BlockSpec((1, B, H), lambda g, i: (g, 0, 0)),
        pl.BlockSpec((1, NPAT, B, H), lambda g, i: (g, 0, 0, 0)),
    ],
    out_shape=[
        jax.ShapeDtypeStruct((2, B, H), jnp.float32),
        jax.ShapeDtypeStruct((2, NPAT, B, H), jnp.float32),
    ],
    scratch_shapes=[
        pltpu.VMEM((B, H), jnp.float32),
        pltpu.VMEM((NPAT, B, H), jnp.float32),
    ],
)


def _tc_final_body(outsum_ref, pool_ref, pemb3_ref, ddi_ref,
                   w0a_ref, w0b_ref, w0c_ref, w0d_ref, bm0_ref,
                   wm1_ref, bm1_ref, wm2_ref, bm2_ref, wout_ref, bout_ref,
                   score_ref):
    pemb = pemb3_ref[...]
    pools = []
    for g in range(2):
        po = pool_ref[...][g] + pemb
        nsq = jnp.sum(po * po, axis=-1, keepdims=True)
        den = jnp.maximum(jnp.sqrt(nsq), 1e-12)
        pools.append(po / den)
    p1n, p2n = pools

    pieces = []
    for p in range(NPAT):
        pieces.append(jnp.sum(p2n * p1n[p][None], axis=-1))
    sim_t = jnp.concatenate(pieces, axis=0)

    ddi = ddi_ref[...][0]
    tid = lax.broadcasted_iota(jnp.int32, (B, T), 1)
    onehot = (tid == ddi[:, None]).astype(jnp.float32)

    outs = outsum_ref[...]
    h = (jnp.dot(outs[0], w0a_ref[...], preferred_element_type=jnp.float32)
         + jnp.dot(outs[1], w0b_ref[...], preferred_element_type=jnp.float32)
         + lax.dot_general(sim_t, w0c_ref[...],
                           (((0,), (0,)), ((), ())),
                           preferred_element_type=jnp.float32)
         + jnp.dot(onehot, w0d_ref[...], preferred_element_type=jnp.float32)
         + bm0_ref[...])
    h = jnp.maximum(jnp.dot(h, wm1_ref[...],
                            preferred_element_type=jnp.float32)
                    + bm1_ref[...], 0.0)
    h = jnp.maximum(jnp.dot(h, wm2_ref[...],
                            preferred_element_type=jnp.float32)
                    + bm2_ref[...], 0.0)
    score_ref[...] = jnp.dot(h, wout_ref[...],
                             preferred_element_type=jnp.float32) + bout_ref[...]


_tc_final = pl.pallas_call(
    _tc_final_body,
    out_shape=jax.ShapeDtypeStruct((B, 1), jnp.float32),
)


def kernel(x1, edge_index1, batch1, x2, edge_index2, batch2, ddi_type,
           W_fc, b_fc, W_g1, b_g1, W_g2, b_g2, P_emb,
           W_m0, b_m0, W_m1, b_m1, W_m2, b_m2, W_out, b_out):
    f32 = jnp.float32
    dst1 = edge_index1[1].reshape(NW, NCHUNK, CH)
    dst2 = edge_index2[1].reshape(NW, NCHUNK, CH)
    e1 = edge_index1.reshape(2, NW, NCHUNK, CH).transpose(1, 2, 0, 3)
    e2 = edge_index2.reshape(2, NW, NCHUNK, CH).transpose(1, 2, 0, 3)

    ones1 = jnp.ones((CH,), f32)
    zeros1 = jnp.zeros((NPAD1,), f32)
    zerosNH = jnp.zeros((N, H), f32)

    deg_k, edge_k = _sc_kernels()
    dp1, dp2 = deg_k(dst1, dst2, ones1, zeros1)

    ht1_1, ht1_2, dinv1, dinv2 = _tc_prep(
        x1, x2, dp1.T[:N], dp2.T[:N], W_fc, b_fc.reshape(1, H), W_g1)

    p1, p2 = edge_k(ht1_1, e1, ht1_2, e2, zerosNH)

    ht2_1, ht2_2 = _tc_mid(p1, p2, ht1_1, ht1_2, dinv1, dinv2,
                           b_g1.reshape(1, H), W_g2)

    q1, q2 = edge_k(ht2_1, e1, ht2_2, e2, zerosNH)

    batr = jnp.stack([batch1, batch2]).reshape(2 * NG, 1, RB)
    outsum, pool = _tc_pool(
        jnp.stack([q1, q2]), jnp.stack([ht2_1, ht2_2]),
        jnp.stack([dinv1, dinv2]), batr,
        b_g2.reshape(1, H), P_emb.T)

    score = _tc_final(
        outsum, pool, P_emb[:, None, :], ddi_type.reshape(1, B),
        W_m0[0:H], W_m0[H:2 * H], W_m0[2 * H:2 * H + NPAT * NPAT],
        W_m0[2 * H + NPAT * NPAT:], b_m0.reshape(1, H),
        W_m1, b_m1.reshape(1, H), W_m2, b_m2.reshape(1, H),
        W_out, b_out.reshape(1, 1))
    return score[:, 0]


# speedup vs baseline: 26.8437x; 1.0462x over previous
import functools

import jax
import jax.numpy as jnp
from jax import lax
from jax.experimental import pallas as pl
from jax.experimental.pallas import tpu as pltpu
from jax.experimental.pallas import tpu_sc as plsc

N = 10000
E = 320000
D = 128
H = 128
NPAT = 16
B = 256
T = 86

NC = 2
NS = 16
NW = NC * NS
PER_TILE = E // NW
CH = 80
NCHUNK = PER_TILE // CH
SLAB = 624
TAIL_START = SLAB * NS
TAIL = N - TAIL_START

NPAD1 = 10240
SLAB1 = NPAD1 // NS


def _tile_rows_copy(src_ref, dst_ref, s):
    start = pl.multiple_of(s * SLAB, 8)
    pltpu.sync_copy(src_ref.at[pl.ds(start, SLAB)],
                    dst_ref.at[pl.ds(start, SLAB)])

    @pl.when(s == NS - 1)
    def _():
        pltpu.sync_copy(src_ref.at[pl.ds(TAIL_START, TAIL)],
                        dst_ref.at[pl.ds(TAIL_START, TAIL)])


def _tile_rows_copy_1d(src_ref, dst_ref, s):
    start = pl.multiple_of(s * SLAB1, 128)
    pltpu.sync_copy(src_ref.at[pl.ds(start, SLAB1)],
                    dst_ref.at[pl.ds(start, SLAB1)])


def _deg_body(e1_hbm, e2_hbm, ones_hbm, zeros_hbm, out1, out2,
              acc, i0, i1, i2, i3, ones_v,
              is0, is1, is2, is3, s0, s1, s2, s3):
    c = lax.axis_index("c")
    s = lax.axis_index("s")
    wid = c * NS + s
    idxb = (i0, i1, i2, i3)
    isem = (is0, is1, is2, is3)
    ssem = (s0, s1, s2, s3)
    pltpu.sync_copy(ones_hbm, ones_v)

    def run_graph(e_hbm, out_hbm):
        _tile_rows_copy_1d(zeros_hbm, acc, s)
        plsc.subcore_barrier()

        def idx_start(j, q):
            pltpu.async_copy(e_hbm.at[wid, j], idxb[q], isem[q])

        def idx_wait(j, q):
            pltpu.make_async_copy(e_hbm.at[wid, j], idxb[q], isem[q]).wait()

        def scat_start(u):
            pltpu.async_copy(ones_v, acc.at[idxb[u].at[1]], ssem[u],
                             add=True)

        def scat_wait(u):
            pltpu.make_async_copy(ones_v, acc.at[idxb[u].at[1]],
                                  ssem[u]).wait()

        idx_start(0, 0)
        idx_start(1, 1)

        @pl.loop(0, NCHUNK - 1, step=4)
        def _steady(i):
            for u in range(4):
                j = i + u
                u2 = (u + 2) % 4
                idx_wait(j, u)

                @pl.when(j >= 2)
                def _drain_prev():
                    scat_wait(u2)

                @pl.when(j + 2 < NCHUNK)
                def _reload():
                    idx_start(j + 2, u2)

                scat_start(u)

        idx_wait(NCHUNK - 1, 0)
        scat_wait(2)
        scat_start(0)
        scat_wait(3)
        scat_wait(0)

        plsc.subcore_barrier()
        _tile_rows_copy_1d(acc, out_hbm.at[c], s)

    run_graph(e1_hbm, out1)
    run_graph(e2_hbm, out2)


def _edge_body(ht_hbm, e_hbm, zeros_hbm, out_hbm,
               acc, i0, i1, i2, i3, d0, d1, r0, r1, r2, r3,
               is0, is1, is2, is3, g0, g1, g2, g3, s0, s1, s2, s3):
    c = lax.axis_index("c")
    s = lax.axis_index("s")
    wid = c * NS + s
    idxb = (i0, i1, i2, i3)
    dsti = (d0, d1)
    rows = (r0, r1, r2, r3)
    isem = (is0, is1, is2, is3)
    gsem = (g0, g1, g2, g3)
    ssem = (s0, s1, s2, s3)

    _tile_rows_copy(zeros_hbm, acc, s)
    plsc.subcore_barrier()

    def idx_start(j, q):
        pltpu.async_copy(e_hbm.at[wid, j], idxb[q], isem[q])

    def idx_wait(j, q):
        pltpu.make_async_copy(e_hbm.at[wid, j], idxb[q], isem[q]).wait()

    def gather_start(u):
        pltpu.async_copy(ht_hbm.at[idxb[u].at[0]], rows[u], gsem[u])

    def gather_wait(u):
        pltpu.make_async_copy(ht_hbm.at[idxb[u].at[0]], rows[u],
                              gsem[u]).wait()

    def copy_dst(u, w):
        for k in range(CH // 16):
            dsti[w][0, pl.ds(16 * k, 16)] = idxb[u][1, pl.ds(16 * k, 16)]

    def scat_start(u, w):
        pltpu.async_copy(rows[u], acc.at[dsti[w].at[0]], ssem[u], add=True)

    def scat_wait(u, w):
        pltpu.make_async_copy(rows[u], acc.at[dsti[w].at[0]],
                              ssem[u]).wait()

    for q in range(4):
        idx_start(q, q)
    idx_wait(0, 0)
    gather_start(0)
    idx_wait(1, 1)
    gather_start(1)

    @pl.loop(0, NCHUNK - 1, step=4)
    def _steady(i):
        for u in range(4):
            j = i + u
            w = u % 2
            u2 = (u + 2) % 4
            gather_wait(u)

            @pl.when(j >= 2)
            def _drain_prev():
                scat_wait(u2, w)

            copy_dst(u, w)
            scat_start(u, w)

            @pl.when(j + 2 < NCHUNK)
            def _refill():
                idx_wait(j + 2, u2)
                gather_start(u2)

            @pl.when(j + 4 < NCHUNK)
            def _reload():
                idx_start(j + 4, u)

    gather_wait(0)
    scat_wait(2, 0)
    copy_dst(0, 0)
    scat_start(0, 0)
    scat_wait(3, 1)
    scat_wait(0, 0)

    plsc.subcore_barrier()
    _tile_rows_copy(acc, out_hbm.at[c], s)


@functools.lru_cache(maxsize=None)
def _sc_kernels():
    mesh = plsc.VectorSubcoreMesh(core_axis_name="c", subcore_axis_name="s",
                                  num_cores=NC, num_subcores=NS)
    deg = pl.kernel(
        _deg_body,
        out_type=(
            jax.ShapeDtypeStruct((NC, NPAD1), jnp.float32),
            jax.ShapeDtypeStruct((NC, NPAD1), jnp.float32),
        ),
        mesh=mesh,
        scratch_types=[
            pltpu.VMEM_SHARED((NPAD1,), jnp.float32),
            pltpu.VMEM((2, CH), jnp.int32),
            pltpu.VMEM((2, CH), jnp.int32),
            pltpu.VMEM((2, CH), jnp.int32),
            pltpu.VMEM((2, CH), jnp.int32),
            pltpu.VMEM((CH,), jnp.float32),
        ] + [pltpu.SemaphoreType.DMA] * 8,
    )
    edge = pl.kernel(
        _edge_body,
        out_type=jax.ShapeDtypeStruct((NC, N, H), jnp.float32),
        mesh=mesh,
        scratch_types=[
            pltpu.VMEM_SHARED((N, H), jnp.float32),
            pltpu.VMEM((2, CH), jnp.int32),
            pltpu.VMEM((2, CH), jnp.int32),
            pltpu.VMEM((2, CH), jnp.int32),
            pltpu.VMEM((2, CH), jnp.int32),
            pltpu.VMEM((1, CH), jnp.int32),
            pltpu.VMEM((1, CH), jnp.int32),
            pltpu.VMEM((CH, H), jnp.float32),
            pltpu.VMEM((CH, H), jnp.float32),
            pltpu.VMEM((CH, H), jnp.float32),
            pltpu.VMEM((CH, H), jnp.float32),
        ] + [pltpu.SemaphoreType.DMA] * 12,
    )
    return deg, edge


RB = 1000
NG = N // RB


def _tc_prep_body(x_ref, dp_ref, wfc_ref, bfc_ref, wg1_ref,
                  ht_ref, dinv_ref, weff_s, beff_s):
    i = pl.program_id(0)

    @pl.when(i == 0)
    def _():
        weff_s[...] = jnp.dot(wfc_ref[...], wg1_ref[...],
                              preferred_element_type=jnp.float32)
        beff_s[...] = jnp.dot(bfc_ref[...], wg1_ref[...],
                              preferred_element_type=jnp.float32)

    dp = dp_ref[...]
    deg = jnp.maximum(dp[:, 0:1] + dp[:, 1:2] + 1.0, 1.0)
    dinv = lax.rsqrt(deg)
    dinv_ref[...] = jnp.broadcast_to(dinv, (RB, 8))
    h = jnp.dot(x_ref[...], weff_s[...],
                preferred_element_type=jnp.float32) + beff_s[...]
    ht_ref[...] = dinv * h


_tc_prep = pl.pallas_call(
    _tc_prep_body,
    grid=(NG,),
    in_specs=[
        pl.BlockSpec((RB, D), lambda i: (i, 0)),
        pl.BlockSpec((RB, NC), lambda i: (i, 0)),
        pl.BlockSpec((D, H), lambda i: (0, 0)),
        pl.BlockSpec((1, H), lambda i: (0, 0)),
        pl.BlockSpec((H, H), lambda i: (0, 0)),
    ],
    out_specs=[
        pl.BlockSpec((RB, H), lambda i: (i, 0)),
        pl.BlockSpec((RB, 8), lambda i: (i, 0)),
    ],
    out_shape=[
        jax.ShapeDtypeStruct((N, H), jnp.float32),
        jax.ShapeDtypeStruct((N, 8), jnp.float32),
    ],
    scratch_shapes=[
        pltpu.VMEM((D, H), jnp.float32),
        pltpu.VMEM((1, H), jnp.float32),
    ],
)


def _tc_mid_body(p_ref, ht1_ref, dinv_ref, bg1_ref, wg2_ref, ht2_ref):
    dinv = dinv_ref[...][:, 0:1]
    p = p_ref[...]
    h1 = dinv * (p[0] + p[1] + ht1_ref[...]) + bg1_ref[...]
    ht2_ref[...] = dinv * jnp.dot(h1, wg2_ref[...],
                                  preferred_element_type=jnp.float32)


_tc_mid = pl.pallas_call(
    _tc_mid_body,
    grid=(NG,),
    in_specs=[
        pl.BlockSpec((2, RB, H), lambda i: (0, i, 0)),
        pl.BlockSpec((RB, H), lambda i: (i, 0)),
        pl.BlockSpec((RB, 8), lambda i: (i, 0)),
        pl.BlockSpec((1, H), lambda i: (0, 0)),
        pl.BlockSpec((H, H), lambda i: (0, 0)),
    ],
    out_specs=pl.BlockSpec((RB, H), lambda i: (i, 0)),
    out_shape=jax.ShapeDtypeStruct((N, H), jnp.float32),
)


def _tc_pool_body(q_ref, ht2_ref, dinv_ref, bat_ref, bg2_ref, pembt_ref,
                  outsum_ref, pool_ref, acc):
    i = pl.program_id(0)

    @pl.when(i == 0)
    def _():
        acc[...] = jnp.zeros_like(acc)

    q = q_ref[...]
    dinv = dinv_ref[...][:, 0:1]
    h2 = dinv * (q[0] + q[1] + ht2_ref[...]) + bg2_ref[...]
    scores = jnp.dot(h2, pembt_ref[...],
                     preferred_element_type=jnp.float32)
    m = jnp.max(scores, axis=-1, keepdims=True)
    e = jnp.exp(scores - m)
    a = e / jnp.sum(e, axis=-1, keepdims=True)

    wf = jnp.concatenate(
        [h2] + [a[:, p:p + 1] * h2 for p in range(NPAT)], axis=1)

    bt = bat_ref[...][0]
    rowid = lax.broadcasted_iota(jnp.int32, (B, RB), 0)
    onehot_t = (rowid == bt).astype(jnp.float32)

    acc[...] += jnp.dot(onehot_t, wf, preferred_element_type=jnp.float32)

    @pl.when(i == NG - 1)
    def _():
        av = acc[...]
        outsum_ref[...] = av[:, :H]
        for p in range(NPAT):
            pool_ref[p] = av[:, (p + 1) * H:(p + 2) * H]


_tc_pool = pl.pallas_call(
    _tc_pool_body,
    grid=(NG,),
    in_specs=[
        pl.BlockSpec((2, RB, H), lambda i: (0, i, 0)),
        pl.BlockSpec((RB, H), lambda i: (i, 0)),
        pl.BlockSpec((RB, 8), lambda i: (i, 0)),
        pl.BlockSpec((1, 1, RB), lambda i: (i, 0, 0)),
        pl.BlockSpec((1, H), lambda i: (0, 0)),
        pl.BlockSpec((H, NPAT), lambda i: (0, 0)),
    ],
    out_specs=[
        pl.BlockSpec((B, H), lambda i: (0, 0)),
        pl.BlockSpec((NPAT, B, H), lambda i: (0, 0, 0)),
    ],
    out_shape=[
        jax.ShapeDtypeStruct((B, H), jnp.float32),
        jax.ShapeDtypeStruct((NPAT, B, H), jnp.float32),
    ],
    scratch_shapes=[
        pltpu.VMEM((B, (NPAT + 1) * H), jnp.float32),
    ],
)


def _tc_final_body(outs1_ref, outs2_ref, pool1_ref, pool2_ref, pemb3_ref,
                   ddi_ref, w0a_ref, w0b_ref, w0c_ref, w0d_ref, bm0_ref,
                   wm1_ref, bm1_ref, wm2_ref, bm2_ref, wout_ref, bout_ref,
                   score_ref):
    pemb = pemb3_ref[...]
    pools = []
    for pref in (pool1_ref, pool2_ref):
        po = pref[...] + pemb
        nsq = jnp.sum(po * po, axis=-1, keepdims=True)
        den = jnp.maximum(jnp.sqrt(nsq), 1e-12)
        pools.append(po / den)
    p1n, p2n = pools

    pieces = []
    for p in range(NPAT):
        pieces.append(jnp.sum(p2n * p1n[p][None], axis=-1))
    sim_t = jnp.concatenate(pieces, axis=0)

    ddi = ddi_ref[...][0]
    tid = lax.broadcasted_iota(jnp.int32, (B, T), 1)
    onehot = (tid == ddi[:, None]).astype(jnp.float32)

    h = (jnp.dot(outs1_ref[...], w0a_ref[...],
                 preferred_element_type=jnp.float32)
         + jnp.dot(outs2_ref[...], w0b_ref[...],
                   preferred_element_type=jnp.float32)
         + lax.dot_general(sim_t, w0c_ref[...],
                           (((0,), (0,)), ((), ())),
                           preferred_element_type=jnp.float32)
         + jnp.dot(onehot, w0d_ref[...], preferred_element_type=jnp.float32)
         + bm0_ref[...])
    h = jnp.maximum(jnp.dot(h, wm1_ref[...],
                            preferred_element_type=jnp.float32)
                    + bm1_ref[...], 0.0)
    h = jnp.maximum(jnp.dot(h, wm2_ref[...],
                            preferred_element_type=jnp.float32)
                    + bm2_ref[...], 0.0)
    score_ref[...] = jnp.dot(h, wout_ref[...],
                             preferred_element_type=jnp.float32) + bout_ref[...]


_tc_final = pl.pallas_call(
    _tc_final_body,
    out_shape=jax.ShapeDtypeStruct((B, 1), jnp.float32),
)


def kernel(x1, edge_index1, batch1, x2, edge_index2, batch2, ddi_type,
           W_fc, b_fc, W_g1, b_g1, W_g2, b_g2, P_emb,
           W_m0, b_m0, W_m1, b_m1, W_m2, b_m2, W_out, b_out):
    f32 = jnp.float32
    e1 = edge_index1.reshape(2, NW, NCHUNK, CH).transpose(1, 2, 0, 3)
    e2 = edge_index2.reshape(2, NW, NCHUNK, CH).transpose(1, 2, 0, 3)

    ones1 = jnp.ones((CH,), f32)
    zeros1 = jnp.zeros((NPAD1,), f32)
    zerosNH = jnp.zeros((N, H), f32)

    deg_k, edge_k = _sc_kernels()
    dp1, dp2 = deg_k(e1, e2, ones1, zeros1)

    bfc2 = b_fc.reshape(1, H)
    ht1_1, dinv1 = _tc_prep(x1, dp1.T[:N], W_fc, bfc2, W_g1)
    ht1_2, dinv2 = _tc_prep(x2, dp2.T[:N], W_fc, bfc2, W_g1)

    p1 = edge_k(ht1_1, e1, zerosNH)
    p2 = edge_k(ht1_2, e2, zerosNH)

    bg12 = b_g1.reshape(1, H)
    ht2_1 = _tc_mid(p1, ht1_1, dinv1, bg12, W_g2)
    ht2_2 = _tc_mid(p2, ht1_2, dinv2, bg12, W_g2)

    q1 = edge_k(ht2_1, e1, zerosNH)
    q2 = edge_k(ht2_2, e2, zerosNH)

    bg22 = b_g2.reshape(1, H)
    pembt = P_emb.T
    os1, pool1 = _tc_pool(q1, ht2_1, dinv1, batch1.reshape(NG, 1, RB),
                          bg22, pembt)
    os2, pool2 = _tc_pool(q2, ht2_2, dinv2, batch2.reshape(NG, 1, RB),
                          bg22, pembt)

    score = _tc_final(
        os1, os2, pool1, pool2, P_emb[:, None, :], ddi_type.reshape(1, B),
        W_m0[0:H], W_m0[H:2 * H], W_m0[2 * H:2 * H + NPAT * NPAT],
        W_m0[2 * H + NPAT * NPAT:], b_m0.reshape(1, H),
        W_m1, b_m1.reshape(1, H), W_m2, b_m2.reshape(1, H),
        W_out, b_out.reshape(1, 1))
    return score[:, 0]


# speedup vs baseline: 28.0341x; 1.0443x over previous
import functools

import jax
import jax.numpy as jnp
from jax import lax
from jax.experimental import pallas as pl
from jax.experimental.pallas import tpu as pltpu
from jax.experimental.pallas import tpu_sc as plsc

N = 10000
E = 320000
D = 128
H = 128
NPAT = 16
B = 256
T = 86

NC = 2
NS = 16
NW = NC * NS
PER_TILE = E // NW
CH = 80
NCHUNK = PER_TILE // CH
SLAB = 624
TAIL_START = SLAB * NS
TAIL = N - TAIL_START

NPAD1 = 10240
SLAB1 = NPAD1 // NS


def _tile_rows_copy(src_ref, dst_ref, s):
    start = pl.multiple_of(s * SLAB, 8)
    pltpu.sync_copy(src_ref.at[pl.ds(start, SLAB)],
                    dst_ref.at[pl.ds(start, SLAB)])

    @pl.when(s == NS - 1)
    def _():
        pltpu.sync_copy(src_ref.at[pl.ds(TAIL_START, TAIL)],
                        dst_ref.at[pl.ds(TAIL_START, TAIL)])


def _tile_rows_copy_1d(src_ref, dst_ref, s):
    start = pl.multiple_of(s * SLAB1, 128)
    pltpu.sync_copy(src_ref.at[pl.ds(start, SLAB1)],
                    dst_ref.at[pl.ds(start, SLAB1)])


def _deg_body(dst1_hbm, dst2_hbm, ones_hbm, zeros_hbm, out1, out2,
              acc, idx_all, ones_v, ssem):
    c = lax.axis_index("c")
    s = lax.axis_index("s")
    wid = c * NS + s
    pltpu.sync_copy(ones_hbm, ones_v)

    def run_graph(dst_hbm, out_hbm):
        _tile_rows_copy_1d(zeros_hbm, acc, s)
        pltpu.sync_copy(dst_hbm.at[wid], idx_all)
        plsc.subcore_barrier()

        @pl.loop(0, NCHUNK, step=5)
        def _chunks(i):
            for u in range(5):
                pltpu.async_copy(ones_v, acc.at[idx_all.at[i + u]], ssem,
                                 add=True)
            for u in range(5):
                pltpu.make_async_copy(ones_v, acc.at[idx_all.at[i + u]],
                                      ssem).wait()

        plsc.subcore_barrier()
        _tile_rows_copy_1d(acc, out_hbm.at[c], s)

    run_graph(dst1_hbm, out1)
    run_graph(dst2_hbm, out2)


def _edge_body(ht_hbm, e_hbm, zeros_hbm, out_hbm,
               acc, i0, i1, i2, i3, d0, d1, r0, r1, r2, r3,
               is0, is1, is2, is3, g0, g1, g2, g3, s0, s1, s2, s3):
    c = lax.axis_index("c")
    s = lax.axis_index("s")
    wid = c * NS + s
    idxb = (i0, i1, i2, i3)
    dsti = (d0, d1)
    rows = (r0, r1, r2, r3)
    isem = (is0, is1, is2, is3)
    gsem = (g0, g1, g2, g3)
    ssem = (s0, s1, s2, s3)

    _tile_rows_copy(zeros_hbm, acc, s)
    plsc.subcore_barrier()

    def idx_start(j, q):
        pltpu.async_copy(e_hbm.at[wid, j], idxb[q], isem[q])

    def idx_wait(j, q):
        pltpu.make_async_copy(e_hbm.at[wid, j], idxb[q], isem[q]).wait()

    def gather_start(u):
        pltpu.async_copy(ht_hbm.at[idxb[u].at[0]], rows[u], gsem[u])

    def gather_wait(u):
        pltpu.make_async_copy(ht_hbm.at[idxb[u].at[0]], rows[u],
                              gsem[u]).wait()

    def copy_dst(u, w):
        for k in range(CH // 16):
            dsti[w][0, pl.ds(16 * k, 16)] = idxb[u][1, pl.ds(16 * k, 16)]

    def scat_start(u, w):
        pltpu.async_copy(rows[u], acc.at[dsti[w].at[0]], ssem[u], add=True)

    def scat_wait(u, w):
        pltpu.make_async_copy(rows[u], acc.at[dsti[w].at[0]],
                              ssem[u]).wait()

    for q in range(4):
        idx_start(q, q)
    idx_wait(0, 0)
    gather_start(0)
    idx_wait(1, 1)
    gather_start(1)

    @pl.loop(0, NCHUNK - 1, step=4)
    def _steady(i):
        for u in range(4):
            j = i + u
            w = u % 2
            u2 = (u + 2) % 4
            gather_wait(u)

            @pl.when(j >= 2)
            def _drain_prev():
                scat_wait(u2, w)

            copy_dst(u, w)
            scat_start(u, w)

            @pl.when(j + 2 < NCHUNK)
            def _refill():
                idx_wait(j + 2, u2)
                gather_start(u2)

            @pl.when(j + 4 < NCHUNK)
            def _reload():
                idx_start(j + 4, u)

    gather_wait(0)
    scat_wait(2, 0)
    copy_dst(0, 0)
    scat_start(0, 0)
    scat_wait(3, 1)
    scat_wait(0, 0)

    plsc.subcore_barrier()
    _tile_rows_copy(acc, out_hbm.at[c], s)


@functools.lru_cache(maxsize=None)
def _sc_kernels():
    mesh = plsc.VectorSubcoreMesh(core_axis_name="c", subcore_axis_name="s",
                                  num_cores=NC, num_subcores=NS)
    deg = pl.kernel(
        _deg_body,
        out_type=(
            jax.ShapeDtypeStruct((NC, NPAD1), jnp.float32),
            jax.ShapeDtypeStruct((NC, NPAD1), jnp.float32),
        ),
        mesh=mesh,
        scratch_types=[
            pltpu.VMEM_SHARED((NPAD1,), jnp.float32),
            pltpu.VMEM((NCHUNK, CH), jnp.int32),
            pltpu.VMEM((CH,), jnp.float32),
            pltpu.SemaphoreType.DMA,
        ],
    )
    edge = pl.kernel(
        _edge_body,
        out_type=jax.ShapeDtypeStruct((NC, N, H), jnp.float32),
        mesh=mesh,
        scratch_types=[
            pltpu.VMEM_SHARED((N, H), jnp.float32),
            pltpu.VMEM((2, CH), jnp.int32),
            pltpu.VMEM((2, CH), jnp.int32),
            pltpu.VMEM((2, CH), jnp.int32),
            pltpu.VMEM((2, CH), jnp.int32),
            pltpu.VMEM((1, CH), jnp.int32),
            pltpu.VMEM((1, CH), jnp.int32),
            pltpu.VMEM((CH, H), jnp.float32),
            pltpu.VMEM((CH, H), jnp.float32),
            pltpu.VMEM((CH, H), jnp.float32),
            pltpu.VMEM((CH, H), jnp.float32),
        ] + [pltpu.SemaphoreType.DMA] * 12,
    )
    return deg, edge


RB = 1000
NG = N // RB


def _tc_prep_a_body(x1_ref, x2_ref, wfc_ref, bfc_ref, wg1_ref,
                    hw1_ref, hw2_ref, weff_s, beff_s):
    i = pl.program_id(0)

    @pl.when(i == 0)
    def _():
        weff_s[...] = jnp.dot(wfc_ref[...], wg1_ref[...],
                              preferred_element_type=jnp.float32)
        beff_s[...] = jnp.dot(bfc_ref[...], wg1_ref[...],
                              preferred_element_type=jnp.float32)

    hw1_ref[...] = jnp.dot(x1_ref[...], weff_s[...],
                           preferred_element_type=jnp.float32) + beff_s[...]
    hw2_ref[...] = jnp.dot(x2_ref[...], weff_s[...],
                           preferred_element_type=jnp.float32) + beff_s[...]


_tc_prep_a = pl.pallas_call(
    _tc_prep_a_body,
    grid=(NG,),
    in_specs=[
        pl.BlockSpec((RB, D), lambda i: (i, 0)),
        pl.BlockSpec((RB, D), lambda i: (i, 0)),
        pl.BlockSpec((D, H), lambda i: (0, 0)),
        pl.BlockSpec((1, H), lambda i: (0, 0)),
        pl.BlockSpec((H, H), lambda i: (0, 0)),
    ],
    out_specs=[
        pl.BlockSpec((RB, H), lambda i: (i, 0)),
        pl.BlockSpec((RB, H), lambda i: (i, 0)),
    ],
    out_shape=[
        jax.ShapeDtypeStruct((N, H), jnp.float32),
        jax.ShapeDtypeStruct((N, H), jnp.float32),
    ],
    scratch_shapes=[
        pltpu.VMEM((D, H), jnp.float32),
        pltpu.VMEM((1, H), jnp.float32),
    ],
)


def _tc_prep_b_body(hw1_ref, hw2_ref, dp1_ref, dp2_ref,
                    ht1_ref, ht2_ref, dinv1_ref, dinv2_ref):
    for hw_ref, dp_ref, ht_ref, dinv_ref in (
            (hw1_ref, dp1_ref, ht1_ref, dinv1_ref),
            (hw2_ref, dp2_ref, ht2_ref, dinv2_ref)):
        dp = dp_ref[...]
        deg = jnp.maximum(dp[:, 0:1] + dp[:, 1:2] + 1.0, 1.0)
        dinv = lax.rsqrt(deg)
        dinv_ref[...] = jnp.broadcast_to(dinv, (RB, 8))
        ht_ref[...] = dinv * hw_ref[...]


_tc_prep_b = pl.pallas_call(
    _tc_prep_b_body,
    grid=(NG,),
    in_specs=[
        pl.BlockSpec((RB, H), lambda i: (i, 0)),
        pl.BlockSpec((RB, H), lambda i: (i, 0)),
        pl.BlockSpec((RB, NC), lambda i: (i, 0)),
        pl.BlockSpec((RB, NC), lambda i: (i, 0)),
    ],
    out_specs=[
        pl.BlockSpec((RB, H), lambda i: (i, 0)),
        pl.BlockSpec((RB, H), lambda i: (i, 0)),
        pl.BlockSpec((RB, 8), lambda i: (i, 0)),
        pl.BlockSpec((RB, 8), lambda i: (i, 0)),
    ],
    out_shape=[
        jax.ShapeDtypeStruct((N, H), jnp.float32),
        jax.ShapeDtypeStruct((N, H), jnp.float32),
        jax.ShapeDtypeStruct((N, 8), jnp.float32),
        jax.ShapeDtypeStruct((N, 8), jnp.float32),
    ],
)


def _tc_mid_body(p_ref, ht1_ref, dinv_ref, bg1_ref, wg2_ref, ht2_ref):
    dinv = dinv_ref[...][:, 0:1]
    p = p_ref[...]
    h1 = dinv * (p[0] + p[1] + ht1_ref[...]) + bg1_ref[...]
    ht2_ref[...] = dinv * jnp.dot(h1, wg2_ref[...],
                                  preferred_element_type=jnp.float32)


_tc_mid = pl.pallas_call(
    _tc_mid_body,
    grid=(NG,),
    in_specs=[
        pl.BlockSpec((2, RB, H), lambda i: (0, i, 0)),
        pl.BlockSpec((RB, H), lambda i: (i, 0)),
        pl.BlockSpec((RB, 8), lambda i: (i, 0)),
        pl.BlockSpec((1, H), lambda i: (0, 0)),
        pl.BlockSpec((H, H), lambda i: (0, 0)),
    ],
    out_specs=pl.BlockSpec((RB, H), lambda i: (i, 0)),
    out_shape=jax.ShapeDtypeStruct((N, H), jnp.float32),
)


def _tc_pool_body(q_ref, ht2_ref, dinv_ref, bat_ref, bg2_ref, pembt_ref,
                  outsum_ref, pool_ref, acc):
    i = pl.program_id(0)

    @pl.when(i == 0)
    def _():
        acc[...] = jnp.zeros_like(acc)

    q = q_ref[...]
    dinv = dinv_ref[...][:, 0:1]
    h2 = dinv * (q[0] + q[1] + ht2_ref[...]) + bg2_ref[...]
    scores = jnp.dot(h2, pembt_ref[...],
                     preferred_element_type=jnp.float32)
    m = jnp.max(scores, axis=-1, keepdims=True)
    e = jnp.exp(scores - m)
    a = e / jnp.sum(e, axis=-1, keepdims=True)

    wf = jnp.concatenate(
        [h2] + [a[:, p:p + 1] * h2 for p in range(NPAT)], axis=1)

    bt = bat_ref[...][0]
    rowid = lax.broadcasted_iota(jnp.int32, (B, RB), 0)
    onehot_t = (rowid == bt).astype(jnp.float32)

    acc[...] += jnp.dot(onehot_t, wf, preferred_element_type=jnp.float32)

    @pl.when(i == NG - 1)
    def _():
        av = acc[...]
        outsum_ref[...] = av[:, :H]
        for p in range(NPAT):
            pool_ref[p] = av[:, (p + 1) * H:(p + 2) * H]


_tc_pool = pl.pallas_call(
    _tc_pool_body,
    grid=(NG,),
    in_specs=[
        pl.BlockSpec((2, RB, H), lambda i: (0, i, 0)),
        pl.BlockSpec((RB, H), lambda i: (i, 0)),
        pl.BlockSpec((RB, 8), lambda i: (i, 0)),
        pl.BlockSpec((1, 1, RB), lambda i: (i, 0, 0)),
        pl.BlockSpec((1, H), lambda i: (0, 0)),
        pl.BlockSpec((H, NPAT), lambda i: (0, 0)),
    ],
    out_specs=[
        pl.BlockSpec((B, H), lambda i: (0, 0)),
        pl.BlockSpec((NPAT, B, H), lambda i: (0, 0, 0)),
    ],
    out_shape=[
        jax.ShapeDtypeStruct((B, H), jnp.float32),
        jax.ShapeDtypeStruct((NPAT, B, H), jnp.float32),
    ],
    scratch_shapes=[
        pltpu.VMEM((B, (NPAT + 1) * H), jnp.float32),
    ],
)


def _tc_final_body(outs1_ref, outs2_ref, pool1_ref, pool2_ref, pemb3_ref,
                   ddi_ref, w0a_ref, w0b_ref, w0c_ref, w0d_ref, bm0_ref,
                   wm1_ref, bm1_ref, wm2_ref, bm2_ref, wout_ref, bout_ref,
                   score_ref):
    pemb = pemb3_ref[...]
    pools = []
    for pref in (pool1_ref, pool2_ref):
        po = pref[...] + pemb
        nsq = jnp.sum(po * po, axis=-1, keepdims=True)
        den = jnp.maximum(jnp.sqrt(nsq), 1e-12)
        pools.append(po / den)
    p1n, p2n = pools

    pieces = []
    for p in range(NPAT):
        pieces.append(jnp.sum(p2n * p1n[p][None], axis=-1))
    sim_t = jnp.concatenate(pieces, axis=0)

    ddi = ddi_ref[...][0]
    tid = lax.broadcasted_iota(jnp.int32, (B, T), 1)
    onehot = (tid == ddi[:, None]).astype(jnp.float32)

    h = (jnp.dot(outs1_ref[...], w0a_ref[...],
                 preferred_element_type=jnp.float32)
         + jnp.dot(outs2_ref[...], w0b_ref[...],
                   preferred_element_type=jnp.float32)
         + lax.dot_general(sim_t, w0c_ref[...],
                           (((0,), (0,)), ((), ())),
                           preferred_element_type=jnp.float32)
         + jnp.dot(onehot, w0d_ref[...], preferred_element_type=jnp.float32)
         + bm0_ref[...])
    h = jnp.maximum(jnp.dot(h, wm1_ref[...],
                            preferred_element_type=jnp.float32)
                    + bm1_ref[...], 0.0)
    h = jnp.maximum(jnp.dot(h, wm2_ref[...],
                            preferred_element_type=jnp.float32)
                    + bm2_ref[...], 0.0)
    score_ref[...] = jnp.dot(h, wout_ref[...],
                             preferred_element_type=jnp.float32) + bout_ref[...]


_tc_final = pl.pallas_call(
    _tc_final_body,
    out_shape=jax.ShapeDtypeStruct((B, 1), jnp.float32),
)


def kernel(x1, edge_index1, batch1, x2, edge_index2, batch2, ddi_type,
           W_fc, b_fc, W_g1, b_g1, W_g2, b_g2, P_emb,
           W_m0, b_m0, W_m1, b_m1, W_m2, b_m2, W_out, b_out):
    f32 = jnp.float32
    e1 = edge_index1.reshape(2, NW, NCHUNK, CH).transpose(1, 2, 0, 3)
    e2 = edge_index2.reshape(2, NW, NCHUNK, CH).transpose(1, 2, 0, 3)
    dst1 = edge_index1[1].reshape(NW, NCHUNK, CH)
    dst2 = edge_index2[1].reshape(NW, NCHUNK, CH)

    ones1 = jnp.ones((CH,), f32)
    zeros1 = jnp.zeros((NPAD1,), f32)
    zerosNH = jnp.zeros((N, H), f32)

    deg_k, edge_k = _sc_kernels()
    dp1, dp2 = deg_k(dst1, dst2, ones1, zeros1)

    hw1, hw2 = _tc_prep_a(x1, x2, W_fc, b_fc.reshape(1, H), W_g1)
    ht1_1, ht1_2, dinv1, dinv2 = _tc_prep_b(hw1, hw2, dp1.T[:N], dp2.T[:N])

    p1 = edge_k(ht1_1, e1, zerosNH)
    p2 = edge_k(ht1_2, e2, zerosNH)

    bg12 = b_g1.reshape(1, H)
    ht2_1 = _tc_mid(p1, ht1_1, dinv1, bg12, W_g2)
    ht2_2 = _tc_mid(p2, ht1_2, dinv2, bg12, W_g2)

    q1 = edge_k(ht2_1, e1, zerosNH)
    q2 = edge_k(ht2_2, e2, zerosNH)

    bg22 = b_g2.reshape(1, H)
    pembt = P_emb.T
    os1, pool1 = _tc_pool(q1, ht2_1, dinv1, batch1.reshape(NG, 1, RB),
                          bg22, pembt)
    os2, pool2 = _tc_pool(q2, ht2_2, dinv2, batch2.reshape(NG, 1, RB),
                          bg22, pembt)

    score = _tc_final(
        os1, os2, pool1, pool2, P_emb[:, None, :], ddi_type.reshape(1, B),
        W_m0[0:H], W_m0[H:2 * H], W_m0[2 * H:2 * H + NPAT * NPAT],
        W_m0[2 * H + NPAT * NPAT:], b_m0.reshape(1, H),
        W_m1, b_m1.reshape(1, H), W_m2, b_m2.reshape(1, H),
        W_out, b_out.reshape(1, 1))
    return score[:, 0]
